# Initial kernel scaffold; baseline (speedup 1.0000x reference)
import jax
import jax.numpy as jnp
from jax.experimental import pallas as pl


def kernel(old_feat, new_feat, target):
    raise NotImplementedError("write your pallas kernel here")


# speedup vs baseline: 11.7911x; 11.7911x over previous
import functools

import jax
import jax.numpy as jnp
from jax.experimental import pallas as pl
from jax.experimental.pallas import tpu as pltpu

_TOPK_POS = 10
_TOPK_NEG = 100
_TEMP = 0.07
_NUM_CLASSES = 100

_U32 = jnp.uint32
_KEY_NEG_INF = 0x007FFFFF


def _float_key(bits):
    flip = jnp.where(bits >= _U32(0x80000000), _U32(0xFFFFFFFF), _U32(0x80000000))
    return bits ^ flip


def _key_to_float(key):
    bits = jnp.where(key >= _U32(0x80000000), key ^ _U32(0x80000000), ~key)
    return jax.lax.bitcast_convert_type(bits, jnp.float32)


def _kth_largest(keys, k):
    rows = keys.shape[0]
    t = jnp.zeros((rows, 1), _U32)
    kf = jnp.float32(k)
    for b in range(31, -1, -1):
        cand = t | _U32(1 << b)
        cnt = jnp.sum((keys >= cand).astype(jnp.float32), axis=1, keepdims=True)
        t = jnp.where(cnt >= kf, cand, t)
    return t


def _body(rows_ref, cols_ref, trow_ref, tcol_ref, out_ref, acc_sum, acc_cnt,
          *, nblocks, kp, kn):
    i = pl.program_id(0)
    rows = rows_ref[...]
    cols = cols_ref[...]
    trow = trow_ref[...]
    tcol = tcol_ref[...]

    row_inv = 1.0 / jnp.maximum(
        jnp.sqrt(jnp.sum(rows * rows, axis=1, keepdims=True)), 1e-12)
    col_inv = 1.0 / jnp.maximum(
        jnp.sqrt(jnp.sum(cols * cols, axis=1, keepdims=True)), 1e-12)
    cols_n = cols * col_inv
    sim = jax.lax.dot_general(
        rows, cols_n, (((1,), (1,)), ((), ())),
        preferred_element_type=jnp.float32)
    sim = sim * row_inv

    pos = trow == tcol

    bits = jax.lax.bitcast_convert_type(sim, _U32)
    key = _float_key(bits)
    keys_neg = jnp.where(pos, _U32(_KEY_NEG_INF), key)
    ikeys_pos = jnp.where(pos, ~key, _U32(_KEY_NEG_INF))

    tn = _kth_largest(keys_neg, kn)
    tp = _kth_largest(ikeys_pos, kp)

    inv_t = jnp.float32(1.0 / _TEMP)
    tn_val = _key_to_float(tn)
    vp = _key_to_float(~tp)

    exp_n = jnp.exp(sim * inv_t)
    gt_n = keys_neg > tn
    cnt_gt = jnp.sum(gt_n.astype(jnp.float32), axis=1, keepdims=True)
    s_neg = (jnp.sum(jnp.where(gt_n, exp_n, 0.0), axis=1, keepdims=True)
             + (jnp.float32(kn) - cnt_gt) * jnp.exp(tn_val * inv_t))

    g = jnp.log(1.0 + s_neg * jnp.exp(-sim * inv_t))
    sel_p = ikeys_pos > tp
    cnt_sel = jnp.sum(sel_p.astype(jnp.float32), axis=1, keepdims=True)
    ties = jnp.float32(kp) - cnt_sel
    g_vp = jnp.log(1.0 + s_neg * jnp.exp(-vp * inv_t))
    lsum = (jnp.sum(jnp.where(sel_p, g, 0.0), axis=1, keepdims=True)
            + ties * g_vp)
    lcnt = (jnp.sum((sel_p & (g != 0.0)).astype(jnp.float32),
                    axis=1, keepdims=True)
            + ties * (g_vp != 0.0).astype(jnp.float32))

    block_sum = jnp.sum(lsum).reshape(1, 1)
    block_cnt = jnp.sum(lcnt).reshape(1, 1)

    @pl.when(i == 0)
    def _():
        acc_sum[...] = jnp.zeros_like(acc_sum)
        acc_cnt[...] = jnp.zeros_like(acc_cnt)

    acc_sum[...] += block_sum
    acc_cnt[...] += block_cnt

    @pl.when(i == nblocks - 1)
    def _():
        out_ref[...] = acc_sum[...] / jnp.maximum(acc_cnt[...], 1.0)


def _run(new_feat, target, *, block_rows=256, interpret=False):
    b, c = new_feat.shape
    kp = min(_TOPK_POS, -(-b // _NUM_CLASSES) - 1, b - 1) if _TOPK_POS > 0 else 1
    kn = min(_TOPK_NEG, b - 1) if _TOPK_NEG > 0 else 1
    tgt = target.astype(jnp.float32)
    nblocks = b // block_rows
    out = pl.pallas_call(
        functools.partial(_body, nblocks=nblocks, kp=kp, kn=kn),
        grid=(nblocks,),
        in_specs=[
            pl.BlockSpec((block_rows, c), lambda i: (i, 0)),
            pl.BlockSpec((b, c), lambda i: (0, 0)),
            pl.BlockSpec((block_rows, 1), lambda i: (i, 0)),
            pl.BlockSpec((1, b), lambda i: (0, 0)),
        ],
        out_specs=pl.BlockSpec((1, 1), lambda i: (0, 0)),
        out_shape=jax.ShapeDtypeStruct((1, 1), jnp.float32),
        scratch_shapes=[pltpu.VMEM((1, 1), jnp.float32),
                        pltpu.VMEM((1, 1), jnp.float32)],
        interpret=interpret,
    )(new_feat, new_feat, tgt.reshape(b, 1), tgt.reshape(1, b))
    return out.reshape(())


def kernel(old_feat, new_feat, target):
    del old_feat
    return _run(new_feat, target)


# speedup vs baseline: 15.0783x; 1.2788x over previous
import functools

import jax
import jax.numpy as jnp
from jax.experimental import pallas as pl
from jax.experimental.pallas import tpu as pltpu

_TOPK_POS = 10
_TOPK_NEG = 100
_TEMP = 0.07
_NUM_CLASSES = 100

_U32 = jnp.uint32
_KEY_NEG_INF = 0x007FFFFF


def _float_key(bits):
    flip = jnp.where(bits >= _U32(0x80000000), _U32(0xFFFFFFFF), _U32(0x80000000))
    return bits ^ flip


def _key_to_float(key):
    bits = jnp.where(key >= _U32(0x80000000), key ^ _U32(0x80000000), ~key)
    return jax.lax.bitcast_convert_type(bits, jnp.float32)


def _kth_largest(keys, k):
    rows = keys.shape[0]
    t = jnp.zeros((rows, 1), _U32)
    kf = jnp.float32(k)
    for b in range(31, -1, -1):
        cand = t | _U32(1 << b)
        cnt = jnp.sum((keys >= cand).astype(jnp.float32), axis=1, keepdims=True)
        t = jnp.where(cnt >= kf, cand, t)
    return t


def _body(rows_ref, cols_ref, trow_ref, tcol_ref, out_ref, acc_sum, acc_cnt,
          *, nblocks, kp, kn):
    i = pl.program_id(0)
    rows = rows_ref[...]
    cols = cols_ref[...]
    trow = trow_ref[...]
    tcol = tcol_ref[...]

    row_inv = 1.0 / jnp.maximum(
        jnp.sqrt(jnp.sum(rows * rows, axis=1, keepdims=True)), 1e-12)
    col_inv = 1.0 / jnp.maximum(
        jnp.sqrt(jnp.sum(cols * cols, axis=1, keepdims=True)), 1e-12)
    cols_n = cols * col_inv
    sim = jax.lax.dot_general(
        rows, cols_n, (((1,), (1,)), ((), ())),
        preferred_element_type=jnp.float32)
    sim = sim * row_inv

    pos = trow == tcol

    bits = jax.lax.bitcast_convert_type(sim, _U32)
    key = _float_key(bits)
    keys_neg = jnp.where(pos, _U32(_KEY_NEG_INF), key)

    tn = _kth_largest(keys_neg, kn)

    inv_t = jnp.float32(1.0 / _TEMP)
    tn_val = _key_to_float(tn)

    exp_n = jnp.exp(sim * inv_t)
    gt_n = keys_neg > tn
    cnt_gt = jnp.sum(gt_n.astype(jnp.float32), axis=1, keepdims=True)
    s_neg = (jnp.sum(jnp.where(gt_n, exp_n, 0.0), axis=1, keepdims=True)
             + (jnp.float32(kn) - cnt_gt) * jnp.exp(tn_val * inv_t))

    masked = jnp.where(pos, sim, jnp.float32(jnp.inf))
    remaining = jnp.full((sim.shape[0], 1), jnp.float32(kp))
    lsum = jnp.zeros_like(remaining)
    lcnt = jnp.zeros_like(remaining)
    for _ in range(kp):
        m = jnp.min(masked, axis=1, keepdims=True)
        eq = masked == m
        ceq = jnp.sum(eq.astype(jnp.float32), axis=1, keepdims=True)
        take = jnp.minimum(remaining, ceq)
        fm = jnp.log(1.0 + s_neg * jnp.exp(-m * inv_t))
        lsum += take * fm
        lcnt += take * (fm != 0.0).astype(jnp.float32)
        masked = jnp.where(eq, jnp.float32(jnp.inf), masked)
        remaining -= take

    block_sum = jnp.sum(lsum).reshape(1, 1)
    block_cnt = jnp.sum(lcnt).reshape(1, 1)

    @pl.when(i == 0)
    def _():
        acc_sum[...] = jnp.zeros_like(acc_sum)
        acc_cnt[...] = jnp.zeros_like(acc_cnt)

    acc_sum[...] += block_sum
    acc_cnt[...] += block_cnt

    @pl.when(i == nblocks - 1)
    def _():
        out_ref[...] = acc_sum[...] / jnp.maximum(acc_cnt[...], 1.0)


def _run(new_feat, target, *, block_rows=256, interpret=False):
    b, c = new_feat.shape
    kp = min(_TOPK_POS, -(-b // _NUM_CLASSES) - 1, b - 1) if _TOPK_POS > 0 else 1
    kn = min(_TOPK_NEG, b - 1) if _TOPK_NEG > 0 else 1
    tgt = target.astype(jnp.float32)
    nblocks = b // block_rows
    out = pl.pallas_call(
        functools.partial(_body, nblocks=nblocks, kp=kp, kn=kn),
        grid=(nblocks,),
        in_specs=[
            pl.BlockSpec((block_rows, c), lambda i: (i, 0)),
            pl.BlockSpec((b, c), lambda i: (0, 0)),
            pl.BlockSpec((block_rows, 1), lambda i: (i, 0)),
            pl.BlockSpec((1, b), lambda i: (0, 0)),
        ],
        out_specs=pl.BlockSpec((1, 1), lambda i: (0, 0)),
        out_shape=jax.ShapeDtypeStruct((1, 1), jnp.float32),
        scratch_shapes=[pltpu.VMEM((1, 1), jnp.float32),
                        pltpu.VMEM((1, 1), jnp.float32)],
        interpret=interpret,
    )(new_feat, new_feat, tgt.reshape(b, 1), tgt.reshape(1, b))
    return out.reshape(())


def kernel(old_feat, new_feat, target):
    del old_feat
    return _run(new_feat, target)
